# Initial kernel scaffold; baseline (speedup 1.0000x reference)
import jax
import jax.numpy as jnp
from jax.experimental import pallas as pl


def kernel(probs, log_probs, lengths, i, k):
    raise NotImplementedError("write your pallas kernel here")


# speedup vs baseline: 1.4599x; 1.4599x over previous
import jax
import jax.numpy as jnp
from jax import lax
from jax.experimental import pallas as pl
from jax.experimental.pallas import tpu as pltpu

_EOS = 3
_NEG_INF = float("-inf")


def _topk_kernel(probs_ref, lp_ref, pen_ref, len_ref, sc_ref, pv_ref, ix_ref):
    pr = probs_ref[0]
    lp = lp_ref[0]
    pen = pen_ref[0]
    done = len_ref[0] != 0
    K, V = pr.shape

    col = lax.broadcasted_iota(jnp.int32, (K, V), 1)
    row = lax.broadcasted_iota(jnp.int32, (K, V), 0)
    p = jnp.where(done, jnp.where(col == _EOS, 0.0, _NEG_INF), pr) + lp
    s = p / pen
    fi = row * V + col

    scores = []
    pvals = []
    idxs = []
    for _ in range(5):
        m = jnp.max(s)
        ci = jnp.min(jnp.where(s == m, fi, jnp.int32(2**31 - 1)))
        hit = fi == ci
        pv = jnp.max(jnp.where(hit, p, _NEG_INF))
        scores.append(m)
        pvals.append(pv)
        idxs.append(ci)
        s = jnp.where(hit, _NEG_INF, s)

    sc_ref[0, 0, :] = jnp.stack(scores)
    pv_ref[0, 0, :] = jnp.stack(pvals)
    ix_ref[0, 0, :] = jnp.stack(idxs)


def kernel(probs, log_probs, lengths, i, k):
    B, K, V = probs.shape
    eff = jnp.where(lengths == 0, i + 1, lengths).astype(jnp.float32)
    pen = jnp.power((5.0 + eff) / 6.0, 0.8)

    lp3 = log_probs.reshape(B, K, 1)
    pen3 = pen.reshape(B, K, 1)
    len3 = lengths.reshape(B, K, 1)

    grid = (B,)
    out_shapes = (
        jax.ShapeDtypeStruct((B, 1, 5), jnp.float32),
        jax.ShapeDtypeStruct((B, 1, 5), jnp.float32),
        jax.ShapeDtypeStruct((B, 1, 5), jnp.int32),
    )
    small_spec = pl.BlockSpec((1, K, 1), lambda b: (b, 0, 0))
    out_spec = pl.BlockSpec((1, 1, 5), lambda b: (b, 0, 0))
    sc, pv, ix = pl.pallas_call(
        _topk_kernel,
        grid=grid,
        in_specs=[
            pl.BlockSpec((1, K, V), lambda b: (b, 0, 0)),
            small_spec,
            small_spec,
            small_spec,
        ],
        out_specs=(out_spec, out_spec, out_spec),
        out_shape=out_shapes,
    )(probs, lp3, pen3, len3)

    best_scores = sc.reshape(B, 5)
    new_log_probs = pv.reshape(B, 5)
    best_idx = ix.reshape(B, 5) + jnp.asarray(k - K, jnp.int32)
    best_beams = best_idx // V
    best_tokens = best_idx % V
    return best_scores, new_log_probs, best_beams, best_tokens


# speedup vs baseline: 1.7329x; 1.1870x over previous
import jax
import jax.numpy as jnp
from jax import lax
from jax.experimental import pallas as pl
from jax.experimental.pallas import tpu as pltpu

_EOS = 3
_NEG_INF = float("-inf")
_IMAX = 2**31 - 1


def _scan_kernel(sp_ref, probs_ref, cv_ref, ci_ref):
    j = pl.program_id(0)
    count = sp_ref[0]

    @pl.when(j < count)
    def _():
        x = probs_ref[0]
        V = x.shape[1]
        fi = lax.broadcasted_iota(jnp.int32, (1, V), 1)
        vals, idxs = [], []
        for _ in range(5):
            m = jnp.max(x)
            ci = jnp.min(jnp.where(x == m, fi, _IMAX))
            vals.append(m)
            idxs.append(ci)
            x = jnp.where(fi == ci, _NEG_INF, x)
        cv_ref[0, 0, :] = jnp.stack(vals)
        ci_ref[0, 0, :] = jnp.stack(idxs)


def _merge_kernel(cv_ref, ci_ref, lp_ref, pen_ref, len_ref,
                  sc_ref, pv_ref, ix_ref):
    cv = cv_ref[...]
    ci = ci_ref[...]
    lp = lp_ref[...]
    pen = pen_ref[...]
    done = len_ref[...] != 0
    B, K, _ = cv.shape
    V = 100000

    krow = lax.broadcasted_iota(jnp.int32, (B, K, 5), 1)
    x_act = jnp.where(done, _NEG_INF, (lp + cv) / pen)
    i_act = jnp.where(done, _IMAX, krow * V + ci)
    p_act = jnp.where(done, _NEG_INF, lp + cv)

    krow1 = lax.broadcasted_iota(jnp.int32, (B, K, 1), 1)
    x_eos = jnp.where(done, lp / pen, _NEG_INF)
    i_eos = jnp.where(done, krow1 * V + _EOS, _IMAX)
    p_eos = jnp.where(done, lp, _NEG_INF)

    X = jnp.concatenate([x_act, x_eos], axis=2)
    I = jnp.concatenate([i_act, i_eos], axis=2)
    P = jnp.concatenate([p_act, p_eos], axis=2)

    scs, pvs, ixs = [], [], []
    for _ in range(5):
        m = jnp.max(jnp.max(X, axis=2), axis=1)
        mb = m[:, None, None]
        c = jnp.min(jnp.min(jnp.where(X == mb, I, _IMAX), axis=2), axis=1)
        cb = c[:, None, None]
        pv = jnp.max(jnp.max(jnp.where(I == cb, P, _NEG_INF), axis=2), axis=1)
        scs.append(m)
        pvs.append(pv)
        ixs.append(c)
        X = jnp.where(I == cb, _NEG_INF, X)

    sc_ref[...] = jnp.stack(scs, axis=1)
    pv_ref[...] = jnp.stack(pvs, axis=1)
    ix_ref[...] = jnp.stack(ixs, axis=1)


def kernel(probs, log_probs, lengths, i, k):
    B, K, V = probs.shape
    R = B * K
    probs2 = probs.reshape(R, 1, V)
    len_flat = lengths.reshape(R)

    active = len_flat == 0
    order = jnp.argsort(jnp.logical_not(active), stable=True).astype(jnp.int32)
    count = jnp.sum(active).astype(jnp.int32)
    last = jnp.take(order, jnp.maximum(count - 1, 0))
    rows = jnp.where(jnp.arange(R, dtype=jnp.int32) < count, order, last)
    sp = jnp.concatenate([count[None], rows])

    cv, ci = pl.pallas_call(
        _scan_kernel,
        grid_spec=pltpu.PrefetchScalarGridSpec(
            num_scalar_prefetch=1,
            grid=(R,),
            in_specs=[
                pl.BlockSpec((1, 1, V), lambda j, sp: (sp[j + 1], 0, 0)),
            ],
            out_specs=(
                pl.BlockSpec((1, 1, 5), lambda j, sp: (sp[j + 1], 0, 0)),
                pl.BlockSpec((1, 1, 5), lambda j, sp: (sp[j + 1], 0, 0)),
            ),
        ),
        out_shape=(
            jax.ShapeDtypeStruct((R, 1, 5), jnp.float32),
            jax.ShapeDtypeStruct((R, 1, 5), jnp.int32),
        ),
    )(sp, probs2)

    eff = jnp.where(lengths == 0, i + 1, lengths).astype(jnp.float32)
    pen = jnp.power((5.0 + eff) / 6.0, 0.8)

    full = lambda s: pl.BlockSpec(s, lambda: (0,) * len(s))
    sc, pv, ix = pl.pallas_call(
        _merge_kernel,
        in_specs=[
            full((B, K, 5)), full((B, K, 5)), full((B, K, 1)),
            full((B, K, 1)), full((B, K, 1)),
        ],
        out_specs=(full((B, 5)), full((B, 5)), full((B, 5))),
        out_shape=(
            jax.ShapeDtypeStruct((B, 5), jnp.float32),
            jax.ShapeDtypeStruct((B, 5), jnp.float32),
            jax.ShapeDtypeStruct((B, 5), jnp.int32),
        ),
    )(cv.reshape(B, K, 5), ci.reshape(B, K, 5), log_probs.reshape(B, K, 1),
      pen.reshape(B, K, 1), lengths.reshape(B, K, 1))

    best_idx = ix + jnp.asarray(k - K, jnp.int32)
    best_beams = best_idx // V
    best_tokens = best_idx % V
    return sc, pv, best_beams, best_tokens
